# Initial kernel scaffold; baseline (speedup 1.0000x reference)
import jax
import jax.numpy as jnp
from jax.experimental import pallas as pl


def kernel(feat, emb_feat):
    raise NotImplementedError("write your pallas kernel here")


# speedup vs baseline: 1.1031x; 1.1031x over previous
import functools

import jax
import jax.numpy as jnp
from jax import lax
from jax.experimental import pallas as pl
from jax.experimental.pallas import tpu as pltpu
from jax.experimental.pallas import tpu_sc as plsc

_VOCAB = 100000
_EMBED = 64
_BATCH = 4096
_FIELDS = 26

_NC = 2
_NS = 16
_NW = _NC * _NS

_TOTAL = _BATCH * _FIELDS
_CHUNK = 128
_NCHUNKS = _TOTAL // (_NW * _CHUNK)
_ROWS_PER_W = _NCHUNKS * _CHUNK


@jax.jit
def _sc_gather(idx2d, table):
  mesh = plsc.VectorSubcoreMesh(core_axis_name="c", subcore_axis_name="s")

  @functools.partial(
      pl.kernel,
      mesh=mesh,
      compiler_params=pltpu.CompilerParams(use_tc_tiling_on_sc=False),
      out_type=jax.ShapeDtypeStruct((_TOTAL, _EMBED), jnp.float32),
      scratch_types=[
          pltpu.VMEM((_NCHUNKS, 1, _CHUNK), jnp.int32),
          pltpu.VMEM((_CHUNK, _EMBED), jnp.float32),
          pltpu.SemaphoreType.DMA,
      ],
  )
  def k(table_hbm, idx_hbm, out_hbm, idx_v, rows_v, sem):
    wid = lax.axis_index("s") * _NC + lax.axis_index("c")
    pltpu.sync_copy(idx_hbm.at[pl.ds(wid * _NCHUNKS, _NCHUNKS)], idx_v)
    base = wid * _ROWS_PER_W

    def body(j, carry):
      pltpu.async_copy(table_hbm.at[idx_v.at[j, 0]], rows_v, sem).wait()
      pltpu.sync_copy(rows_v, out_hbm.at[pl.ds(base + j * _CHUNK, _CHUNK)])
      return carry

    lax.fori_loop(0, _NCHUNKS, body, 0)

  return k(table, idx2d)


def kernel(feat, emb_feat):
  idx2d = feat.reshape(_TOTAL // _CHUNK, 1, _CHUNK)
  out = _sc_gather(idx2d, emb_feat)
  return out.reshape(_BATCH, _FIELDS, _EMBED)


# speedup vs baseline: 1.2195x; 1.1055x over previous
import functools

import jax
import jax.numpy as jnp
from jax import lax
from jax.experimental import pallas as pl
from jax.experimental.pallas import tpu as pltpu
from jax.experimental.pallas import tpu_sc as plsc

_VOCAB = 100000
_EMBED = 64
_BATCH = 4096
_FIELDS = 26

_NC = 2
_NS = 16
_NW = _NC * _NS

_TOTAL = _BATCH * _FIELDS
_CH = 104
_NCH = 32
_NB = 8
_ROUNDS = _NCH // _NB
_RPB = _NB * _CH
_ROWS_PER_W = _NCH * _CH


@jax.jit
def _sc_gather(idx3d, table):
  mesh = plsc.VectorSubcoreMesh(core_axis_name="c", subcore_axis_name="s")

  @functools.partial(
      pl.kernel,
      mesh=mesh,
      compiler_params=pltpu.CompilerParams(use_tc_tiling_on_sc=False),
      out_type=jax.ShapeDtypeStruct((_TOTAL, _EMBED), jnp.float32),
      scratch_types=[
          pltpu.VMEM((_NCH, 1, _CH), jnp.int32),
          pltpu.VMEM((2, _RPB, _EMBED), jnp.float32),
          pltpu.SemaphoreType.DMA,
          pltpu.SemaphoreType.DMA,
      ],
  )
  def k(table_hbm, idx_hbm, out_hbm, idx_v, stage, sem_g, sem_o):
    wid = lax.axis_index("s") * _NC + lax.axis_index("c")
    pltpu.sync_copy(idx_hbm.at[pl.ds(wid * _NCH, _NCH)], idx_v)
    base = wid * _ROWS_PER_W

    def fire_round(r, p):
      for b in range(_NB):
        pltpu.async_copy(
            table_hbm.at[idx_v.at[r * _NB + b, 0]],
            stage.at[p, pl.ds(b * _CH, _CH)],
            sem_g,
        )

    def drain_gather_round(p):
      pltpu.make_async_copy(
          table_hbm.at[pl.ds(0, _RPB)], stage.at[p], sem_g
      ).wait()

    def drain_out_round():
      pltpu.make_async_copy(
          stage.at[0], out_hbm.at[pl.ds(base, _RPB)], sem_o
      ).wait()

    fire_round(0, 0)
    fire_round(1, 1)

    def body(r, c):
      p = lax.rem(r, 2)
      drain_gather_round(p)
      pltpu.async_copy(
          stage.at[p], out_hbm.at[pl.ds(base + r * _RPB, _RPB)], sem_o
      )

      @pl.when(r + 2 < _ROUNDS)
      def _():
        drain_out_round()
        fire_round_dyn(r + 2, p)

      return c

    def fire_round_dyn(r, p):
      for b in range(_NB):
        pltpu.async_copy(
            table_hbm.at[idx_v.at[r * _NB + b, 0]],
            stage.at[p, pl.ds(b * _CH, _CH)],
            sem_g,
        )

    lax.fori_loop(0, _ROUNDS, body, 0)
    drain_out_round()
    drain_out_round()

  return k(table, idx3d)


def kernel(feat, emb_feat):
  idx3d = feat.reshape(_TOTAL // _CH, 1, _CH)
  out = _sc_gather(idx3d, emb_feat)
  return out.reshape(_BATCH, _FIELDS, _EMBED)
